# baseline (speedup 1.0000x reference)
import functools

import jax
import jax.numpy as jnp
from jax import lax
from jax.experimental import pallas as pl
from jax.experimental.pallas import tpu as pltpu
from jax.experimental.pallas import tpu_sc as plsc

NBUF = 2
DEPTH = 1


def kernel(input_ids, embedding_table, pos_emb_table):
    B, L = input_ids.shape
    V, D = embedding_table.shape
    N = B * L
    NW = 32
    LPW = L // NW
    n_per_w = LPW * B
    CHUNK = 64
    n_chunks = n_per_w // CHUNK
    cpl = B // CHUNK
    NS = D // 16
    assert (n_chunks - 2 * DEPTH) % NBUF == 0

    ids_t = input_ids.T.reshape(N).astype(jnp.int32)
    mesh = plsc.VectorSubcoreMesh(core_axis_name="c", subcore_axis_name="s")

    @functools.partial(
        pl.kernel,
        mesh=mesh,
        out_type=jax.ShapeDtypeStruct((N, D), jnp.float32),
        scratch_types=[
            pltpu.VMEM((n_per_w,), jnp.int32),
            pltpu.VMEM((n_chunks, CHUNK), jnp.int32),
            pltpu.VMEM((LPW, D), jnp.float32),
        ] + [pltpu.VMEM((CHUNK, D), jnp.float32)] * NBUF
          + [pltpu.SemaphoreType.DMA] * (2 * NBUF),
    )
    def emb_kernel(ids_hbm, tab_hbm, pos_hbm, out_hbm,
                   idx_v, scat_idx, posbuf, *bufs_and_sems):
        bufs = bufs_and_sems[:NBUF]
        semg = bufs_and_sems[NBUF:2 * NBUF]
        semw = bufs_and_sems[2 * NBUF:]
        wid = lax.axis_index("s") * 2 + lax.axis_index("c")
        l0 = wid * LPW
        base = wid * n_per_w

        H = CHUNK // 2

        def start_gather(c, b):
            off = pl.multiple_of(c * CHUNK, CHUNK)
            pltpu.async_copy(tab_hbm.at[idx_v.at[pl.ds(off, H)]],
                             bufs[b].at[pl.ds(0, H)], semg[b])
            pltpu.async_copy(tab_hbm.at[idx_v.at[pl.ds(off + H, H)]],
                             bufs[b].at[pl.ds(H, H)], semg[b])

        def wait_gather_half(b, h):
            pltpu.make_async_copy(tab_hbm.at[idx_v.at[pl.ds(0, H)]],
                                  bufs[b].at[pl.ds(h * H, H)],
                                  semg[b]).wait()

        def start_write(c, b):
            pltpu.async_copy(bufs[b], out_hbm.at[scat_idx.at[c]], semw[b])

        def wait_write(b):
            pltpu.make_async_copy(bufs[b], out_hbm.at[scat_idx.at[0]],
                                  semw[b]).wait()

        def process(c, b):
            buf = bufs[b]
            lsel = c // cpl
            pos_regs = tuple(posbuf[lsel, pl.ds(i * 16, 16)]
                             for i in range(NS))

            def add_row(r, regs):
                for i in range(NS):
                    s = pl.ds(i * 16, 16)
                    buf[r, s] = buf[r, s] + regs[i]
                return regs

            wait_gather_half(b, 0)
            lax.fori_loop(0, H, add_row, pos_regs)
            wait_gather_half(b, 1)
            lax.fori_loop(H, CHUNK, add_row, pos_regs)
            start_write(c, b)

        pltpu.sync_copy(ids_hbm.at[pl.ds(base, n_per_w)], idx_v)
        for c in range(DEPTH):
            start_gather(c, c)
        pltpu.sync_copy(pos_hbm.at[pl.ds(l0, LPW)], posbuf)

        def build_scat(c, carry):
            lsel = c // cpl
            jj = c - lsel * cpl
            for kk in range(CHUNK // 16):
                v = (jj * CHUNK + kk * 16 + lax.iota(jnp.int32, 16)) * L
                scat_idx[c, pl.ds(kk * 16, 16)] = v + l0 + lsel
            return carry

        lax.fori_loop(0, n_chunks, build_scat, 0)

        for c in range(DEPTH):
            start_gather(c + DEPTH, c + DEPTH)
            process(c, c)

        def group(g, carry):
            for k in range(NBUF):
                c = NBUF * g + DEPTH + k
                gb = (DEPTH + k + DEPTH) % NBUF
                wait_write(gb)
                start_gather(c + DEPTH, gb)
                process(c, (DEPTH + k) % NBUF)
            return carry

        lax.fori_loop(0, (n_chunks - 2 * DEPTH) // NBUF, group, 0)

        for c in range(n_chunks - DEPTH, n_chunks):
            wait_write((c + DEPTH) % NBUF)
            process(c, c % NBUF)
        for c in range(n_chunks - DEPTH, n_chunks):
            wait_write(c % NBUF)

    out = emb_kernel(ids_t, embedding_table, pos_emb_table)
    return out.reshape(B, L, D)


# speedup vs baseline: 1.0045x; 1.0045x over previous
import functools

import jax
import jax.numpy as jnp
from jax import lax
from jax.experimental import pallas as pl
from jax.experimental.pallas import tpu as pltpu
from jax.experimental.pallas import tpu_sc as plsc

NBUF = 2
DEPTH = 1


def kernel(input_ids, embedding_table, pos_emb_table):
    B, L = input_ids.shape
    V, D = embedding_table.shape
    N = B * L
    NW = 32
    LPW = L // NW
    n_per_w = LPW * B
    CHUNK = 64
    n_chunks = n_per_w // CHUNK
    cpl = B // CHUNK
    NS = D // 16
    assert (n_chunks - 2 * DEPTH) % NBUF == 0

    ids_t = input_ids.T.reshape(N).astype(jnp.int32)
    mesh = plsc.VectorSubcoreMesh(core_axis_name="c", subcore_axis_name="s")

    @functools.partial(
        pl.kernel,
        mesh=mesh,
        out_type=jax.ShapeDtypeStruct((N, D), jnp.float32),
        scratch_types=[
            pltpu.VMEM((n_per_w,), jnp.int32),
            pltpu.VMEM((n_chunks, CHUNK), jnp.int32),
            pltpu.VMEM((LPW, D), jnp.float32),
        ] + [pltpu.VMEM((CHUNK, D), jnp.float32)] * NBUF
          + [pltpu.SemaphoreType.DMA] * (2 * NBUF),
    )
    def emb_kernel(ids_hbm, tab_hbm, pos_hbm, out_hbm,
                   idx_v, scat_idx, posbuf, *bufs_and_sems):
        bufs = bufs_and_sems[:NBUF]
        semg = bufs_and_sems[NBUF:2 * NBUF]
        semw = bufs_and_sems[2 * NBUF:]
        wid = lax.axis_index("s") * 2 + lax.axis_index("c")
        l0 = wid * LPW
        base = wid * n_per_w

        def start_gather(c, b):
            off = pl.multiple_of(c * CHUNK, CHUNK)
            pltpu.async_copy(tab_hbm.at[idx_v.at[pl.ds(off, CHUNK)]],
                             bufs[b], semg[b])

        def wait_gather(b):
            pltpu.make_async_copy(tab_hbm.at[idx_v.at[pl.ds(0, CHUNK)]],
                                  bufs[b], semg[b]).wait()

        def start_write(c, b):
            pltpu.async_copy(bufs[b], out_hbm.at[scat_idx.at[c]], semw[b])

        def wait_write(b):
            pltpu.make_async_copy(bufs[b], out_hbm.at[scat_idx.at[0]],
                                  semw[b]).wait()

        def process(c, b):
            buf = bufs[b]
            lsel = c // cpl
            pos_regs = tuple(posbuf[lsel, pl.ds(i * 16, 16)]
                             for i in range(NS))

            def add_row(r, regs):
                for i in range(NS):
                    s = pl.ds(i * 16, 16)
                    buf[r, s] = buf[r, s] + regs[i]
                return regs

            wait_gather(b)
            lax.fori_loop(0, CHUNK, add_row, pos_regs)
            start_write(c, b)

        pltpu.sync_copy(ids_hbm.at[pl.ds(base, n_per_w)], idx_v)
        for c in range(DEPTH):
            start_gather(c, c)
        pltpu.sync_copy(pos_hbm.at[pl.ds(l0, LPW)], posbuf)

        def build_scat(c, carry):
            lsel = c // cpl
            jj = c - lsel * cpl
            for kk in range(CHUNK // 16):
                v = (jj * CHUNK + kk * 16 + lax.iota(jnp.int32, 16)) * L
                scat_idx[c, pl.ds(kk * 16, 16)] = v + l0 + lsel
            return carry

        lax.fori_loop(0, n_chunks, build_scat, 0)

        for c in range(DEPTH):
            start_gather(c + DEPTH, c + DEPTH)
            process(c, c)

        def group(g, carry):
            for k in range(NBUF):
                c = NBUF * g + DEPTH + k
                gb = (DEPTH + k + DEPTH) % NBUF
                wait_write(gb)
                start_gather(c + DEPTH, gb)
                process(c, (DEPTH + k) % NBUF)
            return carry

        lax.fori_loop(0, (n_chunks - 2 * DEPTH) // NBUF, group, 0)

        for c in range(n_chunks - DEPTH, n_chunks):
            wait_write((c + DEPTH) % NBUF)
            process(c, c % NBUF)
        for c in range(n_chunks - DEPTH, n_chunks):
            wait_write(c % NBUF)

    out = emb_kernel(ids_t, embedding_table, pos_emb_table)
    return out.reshape(B, L, D)
